# baseline (speedup 1.0000x reference)
import jax
import jax.numpy as jnp
from jax.experimental import pallas as pl
from jax.experimental.pallas import tpu as pltpu

D = 512
K = 64


def _netvlad_kernel(x0_ref, x1_ref, x2_ref, x3_ref, w_ref, b_ref, c_ref,
                    out_ref):
    w = w_ref[...]
    b = b_ref[...]
    c = c_ref[...]

    vlad = [jnp.zeros((D, K), jnp.float32) for _ in range(2)]
    s = [jnp.zeros((1, K), jnp.float32) for _ in range(2)]
    for x_ref in (x0_ref, x1_ref, x2_ref, x3_ref):
        nq = x_ref.shape[1]
        xh = x_ref[...].reshape(2 * nq, D).astype(jnp.bfloat16)
        logits = jax.lax.dot_general(
            xh, w, (((1,), (1,)), ((), ())),
            preferred_element_type=jnp.float32) + b
        e = jnp.exp(logits)
        alpha = e / jnp.sum(e, axis=1, keepdims=True)
        ab = alpha.astype(jnp.bfloat16)
        for i in range(2):
            s[i] = s[i] + jnp.sum(alpha[i * nq:(i + 1) * nq],
                                  axis=0, keepdims=True)
            vlad[i] = vlad[i] + jax.lax.dot_general(
                xh[i * nq:(i + 1) * nq], ab[i * nq:(i + 1) * nq],
                (((0,), (0,)), ((), ())),
                preferred_element_type=jnp.float32)

    for i in range(2):
        v = vlad[i] - c * s[i]
        ssq = jnp.sum(v * v, axis=0, keepdims=True)
        v = v * jax.lax.rsqrt(ssq)
        gsq = jnp.sum(v * v, axis=(0, 1), keepdims=True)
        out_ref[i] = v * jax.lax.rsqrt(gsq)


def kernel(inputs, conv_w, conv_b, centers):
    B, d, H, W = inputs.shape
    N = H * W
    x = inputs.reshape(B, d, N).transpose(0, 2, 1)
    q = N // 4
    out = pl.pallas_call(
        _netvlad_kernel,
        grid=(B // 2,),
        in_specs=[
            pl.BlockSpec((2, q, d), lambda b, j=j: (b, j, 0)) for j in range(4)
        ] + [
            pl.BlockSpec((K, d), lambda b: (0, 0)),
            pl.BlockSpec((1, K), lambda b: (0, 0)),
            pl.BlockSpec((d, K), lambda b: (0, 0)),
        ],
        out_specs=pl.BlockSpec((2, d, K), lambda b: (b, 0, 0)),
        out_shape=jax.ShapeDtypeStruct((B, d, K), jnp.float32),
        compiler_params=pltpu.CompilerParams(
            dimension_semantics=("arbitrary",),
            vmem_limit_bytes=50 * 1024 * 1024,
        ),
    )(x, x, x, x, conv_w.astype(jnp.bfloat16), conv_b.reshape(1, K), centers)
    return out.reshape(B, d * K)


# speedup vs baseline: 1.1646x; 1.1646x over previous
import jax
import jax.numpy as jnp
from jax.experimental import pallas as pl
from jax.experimental.pallas import tpu as pltpu

D = 512
K = 64


def _netvlad_kernel(x0_ref, x1_ref, w_ref, b_ref, c_ref, out_ref):
    w = w_ref[...]
    b = b_ref[...]
    c = c_ref[...]

    vlad = [jnp.zeros((D, K), jnp.float32) for _ in range(2)]
    s = [jnp.zeros((1, K), jnp.float32) for _ in range(2)]
    for x_ref in (x0_ref, x1_ref):
        nq = x_ref.shape[1]
        xh = x_ref[...].reshape(2 * nq, D).astype(jnp.bfloat16)
        logits = jax.lax.dot_general(
            xh, w, (((1,), (1,)), ((), ())),
            preferred_element_type=jnp.float32) + b
        e = jnp.exp(logits)
        alpha = e / jnp.sum(e, axis=1, keepdims=True)
        ab = alpha.astype(jnp.bfloat16)
        for i in range(2):
            s[i] = s[i] + jnp.sum(alpha[i * nq:(i + 1) * nq],
                                  axis=0, keepdims=True)
            vlad[i] = vlad[i] + jax.lax.dot_general(
                xh[i * nq:(i + 1) * nq], ab[i * nq:(i + 1) * nq],
                (((0,), (0,)), ((), ())),
                preferred_element_type=jnp.float32)

    for i in range(2):
        v = vlad[i] - c * s[i]
        ssq = jnp.sum(v * v, axis=0, keepdims=True)
        v = v * jax.lax.rsqrt(ssq)
        gsq = jnp.sum(v * v, axis=(0, 1), keepdims=True)
        out_ref[i] = v * jax.lax.rsqrt(gsq)


def kernel(inputs, conv_w, conv_b, centers):
    B, d, H, W = inputs.shape
    N = H * W
    x = inputs.reshape(B, d, N).transpose(0, 2, 1)
    h = N // 2
    out = pl.pallas_call(
        _netvlad_kernel,
        grid=(B // 2,),
        in_specs=[
            pl.BlockSpec((2, h, d), lambda b, j=j: (b, j, 0)) for j in range(2)
        ] + [
            pl.BlockSpec((K, d), lambda b: (0, 0)),
            pl.BlockSpec((1, K), lambda b: (0, 0)),
            pl.BlockSpec((d, K), lambda b: (0, 0)),
        ],
        out_specs=pl.BlockSpec((2, d, K), lambda b: (b, 0, 0)),
        out_shape=jax.ShapeDtypeStruct((B, d, K), jnp.float32),
        compiler_params=pltpu.CompilerParams(
            dimension_semantics=("arbitrary",),
            vmem_limit_bytes=50 * 1024 * 1024,
        ),
    )(x, x, conv_w.astype(jnp.bfloat16), conv_b.reshape(1, K), centers)
    return out.reshape(B, d * K)
